# Initial kernel scaffold; baseline (speedup 1.0000x reference)
import jax
import jax.numpy as jnp
from jax.experimental import pallas as pl


def kernel(x, edge_index, edge_type, W1, root1, b1, W2, root2, b2):
    raise NotImplementedError("write your pallas kernel here")


# speedup vs baseline: 7.3093x; 7.3093x over previous
import functools

import jax
import jax.numpy as jnp
from jax import lax
from jax.experimental import pallas as pl
from jax.experimental.pallas import tpu as pltpu
from jax.experimental.pallas import tpu_sc as plsc

N = 10000
E = 320000
R = 8
D_IN = 128
H = 16
C_OUT = 16

NC = 2
NS = 16
NW = NC * NS
CHUNK = 128
CH = -(-E // (NW * CHUNK))
E_PAD = NW * CH * CHUNK

S_ROWS = N * R + CHUNK
C_ROWS = N + 16
S_TRASH = N * R
C_TRASH = N


def _sc_mesh():
    return plsc.VectorSubcoreMesh(
        core_axis_name="c", subcore_axis_name="s", num_cores=NC, num_subcores=NS
    )


def _edge_pass(table, gidx, sidx, zeros, acc_rows):
    stripe = acc_rows // NS

    @functools.partial(
        pl.kernel,
        out_type=jax.ShapeDtypeStruct((NC, acc_rows, 16), jnp.float32),
        mesh=_sc_mesh(),
        scratch_types=[
            pltpu.VMEM_SHARED((acc_rows, 16), jnp.float32),
            pltpu.VMEM((CH, CHUNK), jnp.int32),
            pltpu.VMEM((CH, CHUNK), jnp.int32),
            pltpu.VMEM((CHUNK, 16), jnp.float32),
            pltpu.SemaphoreType.DMA,
        ],
        compiler_params=pltpu.CompilerParams(use_tc_tiling_on_sc=False),
    )
    def kern(table_hbm, gidx_hbm, sidx_hbm, zeros_hbm, out_hbm,
             acc, gidx_v, sidx_v, rows_v, sem):
        c = lax.axis_index("c")
        s = lax.axis_index("s")
        wid = c * NS + s
        pltpu.sync_copy(zeros_hbm.at[pl.ds(s * stripe, stripe)],
                        acc.at[pl.ds(s * stripe, stripe)])
        pltpu.sync_copy(gidx_hbm.at[wid], gidx_v)
        pltpu.sync_copy(sidx_hbm.at[wid], sidx_v)
        plsc.subcore_barrier()

        def body(j, carry):
            pltpu.async_copy(table_hbm.at[gidx_v.at[j]], rows_v, sem).wait()
            pltpu.sync_copy(rows_v, acc.at[sidx_v.at[j]], add=True)
            return carry

        lax.fori_loop(0, CH, body, 0)
        plsc.subcore_barrier()
        pltpu.sync_copy(acc.at[pl.ds(s * stripe, stripe)],
                        out_hbm.at[c, pl.ds(s * stripe, stripe)])

    return kern(table, gidx, sidx, zeros)


def _prep_kernel(x_ref, wcat_ref, root_ref, b_ref, xw_ref, base_ref):
    x = x_ref[...]
    xw_ref[...] = jnp.dot(x, wcat_ref[...], preferred_element_type=jnp.float32)
    base_ref[...] = (
        jnp.dot(x, root_ref[...], preferred_element_type=jnp.float32)
        + b_ref[...]
    )


def _prep(x, wcat, root, b, d_in, blk):
    grid = N // blk
    return pl.pallas_call(
        _prep_kernel,
        grid=(grid,),
        in_specs=[
            pl.BlockSpec((blk, d_in), lambda i: (i, 0)),
            pl.BlockSpec((d_in, R * H), lambda i: (0, 0)),
            pl.BlockSpec((d_in, H), lambda i: (0, 0)),
            pl.BlockSpec((1, H), lambda i: (0, 0)),
        ],
        out_specs=[
            pl.BlockSpec((blk, R * H), lambda i: (i, 0)),
            pl.BlockSpec((blk, H), lambda i: (i, 0)),
        ],
        out_shape=[
            jax.ShapeDtypeStruct((N, R * H), jnp.float32),
            jax.ShapeDtypeStruct((N, H), jnp.float32),
        ],
    )(x, wcat, root, b)


def _agg_message(sa_ref, sb_ref, ca_ref, cb_ref):
    svals = sa_ref[...] + sb_ref[...]
    cnt = ca_ref[...] + cb_ref[...]
    msg = jnp.zeros(sa_ref.shape[:1] + (16,), jnp.float32)
    for r in range(R):
        inv = 1.0 / jnp.maximum(cnt[:, r : r + 1], 1.0)
        msg = msg + svals[:, r * 16 : (r + 1) * 16] * inv
    return msg


def _mid_kernel(sa_ref, sb_ref, ca_ref, cb_ref, base_ref, wcat_ref,
                root_ref, b_ref, hw_ref, base2_ref):
    h = jax.nn.relu(base_ref[...] + _agg_message(sa_ref, sb_ref, ca_ref, cb_ref))
    hw_ref[...] = jnp.dot(h, wcat_ref[...], preferred_element_type=jnp.float32)
    base2_ref[...] = (
        jnp.dot(h, root_ref[...], preferred_element_type=jnp.float32)
        + b_ref[...]
    )


def _final_kernel(sa_ref, sb_ref, ca_ref, cb_ref, base_ref, out_ref):
    out_ref[...] = base_ref[...] + _agg_message(sa_ref, sb_ref, ca_ref, cb_ref)


def kernel(x, edge_index, edge_type, W1, root1, b1, W2, root2, b2):
    src = edge_index[0].astype(jnp.int32)
    dst = edge_index[1].astype(jnp.int32)
    et = edge_type.astype(jnp.int32)

    gidx = src * R + et
    sidx = dst * R + et
    pad = E_PAD - E
    gidx = jnp.concatenate([gidx, jnp.zeros((pad,), jnp.int32)])
    sidx = jnp.concatenate([sidx, jnp.full((pad,), S_TRASH, jnp.int32)])
    cgidx = jnp.concatenate([et, jnp.zeros((pad,), jnp.int32)])
    csidx = jnp.concatenate([dst, jnp.full((pad,), C_TRASH, jnp.int32)])
    gidx = gidx.reshape(NW, CH, CHUNK)
    sidx = sidx.reshape(NW, CH, CHUNK)
    cgidx = cgidx.reshape(NW, CH, CHUNK)
    csidx = csidx.reshape(NW, CH, CHUNK)

    wcat1 = W1.transpose(1, 0, 2).reshape(D_IN, R * H).astype(jnp.float32)
    wcat2 = W2.transpose(1, 0, 2).reshape(H, R * C_OUT).astype(jnp.float32)
    onehot = jnp.eye(R, 16, dtype=jnp.float32)

    zeros_s = jnp.zeros((S_ROWS, 16), jnp.float32)
    zeros_c = jnp.zeros((C_ROWS, 16), jnp.float32)

    cacc = _edge_pass(onehot, cgidx, csidx, zeros_c, C_ROWS)
    ca = cacc[0, :N, :]
    cb = cacc[1, :N, :]

    xw1, base1 = _prep(x, wcat1, root1.astype(jnp.float32),
                       b1.reshape(1, H).astype(jnp.float32), D_IN, 1000)
    s1 = _edge_pass(xw1.reshape(N * R, 16), gidx, sidx, zeros_s, S_ROWS)
    s1a = s1[0, : N * R, :].reshape(N, R * 16)
    s1b = s1[1, : N * R, :].reshape(N, R * 16)

    blk = 1000
    full = lambda shape: pl.BlockSpec(shape, lambda i: (0, 0))
    rowblk = lambda w: pl.BlockSpec((blk, w), lambda i: (i, 0))

    hw2, base2 = pl.pallas_call(
        _mid_kernel,
        grid=(N // blk,),
        in_specs=[
            rowblk(R * 16), rowblk(R * 16), rowblk(16), rowblk(16),
            rowblk(H), full((H, R * C_OUT)), full((H, C_OUT)), full((1, C_OUT)),
        ],
        out_specs=[rowblk(R * C_OUT), rowblk(C_OUT)],
        out_shape=[
            jax.ShapeDtypeStruct((N, R * C_OUT), jnp.float32),
            jax.ShapeDtypeStruct((N, C_OUT), jnp.float32),
        ],
    )(s1a, s1b, ca, cb, base1, wcat2, root2.astype(jnp.float32),
      b2.reshape(1, C_OUT).astype(jnp.float32))

    s2 = _edge_pass(hw2.reshape(N * R, 16), gidx, sidx, zeros_s, S_ROWS)
    s2a = s2[0, : N * R, :].reshape(N, R * 16)
    s2b = s2[1, : N * R, :].reshape(N, R * 16)

    out = pl.pallas_call(
        _final_kernel,
        grid=(N // blk,),
        in_specs=[rowblk(R * 16), rowblk(R * 16), rowblk(16), rowblk(16),
                  rowblk(C_OUT)],
        out_specs=rowblk(C_OUT),
        out_shape=jax.ShapeDtypeStruct((N, C_OUT), jnp.float32),
    )(s2a, s2b, ca, cb, base2)

    return out


# speedup vs baseline: 20.4310x; 2.7952x over previous
import functools

import jax
import jax.numpy as jnp
from jax import lax
from jax.experimental import pallas as pl
from jax.experimental.pallas import tpu as pltpu
from jax.experimental.pallas import tpu_sc as plsc

N = 10000
E = 320000
R = 8
D_IN = 128
H = 16
C_OUT = 16

NC = 2
NS = 16
NW = NC * NS
CHUNK = 128
CH = -(-E // (NW * CHUNK))
E_PAD = NW * CH * CHUNK

S_ROWS = N * R + CHUNK
C_ROWS = N + 16
S_TRASH = N * R
C_TRASH = N


def _sc_mesh():
    return plsc.VectorSubcoreMesh(
        core_axis_name="c", subcore_axis_name="s", num_cores=NC, num_subcores=NS
    )


def _edge_pass(table, gidx, sidx, zeros, acc_rows):
    stripe = acc_rows // NS

    @functools.partial(
        pl.kernel,
        out_type=jax.ShapeDtypeStruct((NC, acc_rows, 16), jnp.float32),
        mesh=_sc_mesh(),
        scratch_types=[
            pltpu.VMEM_SHARED((acc_rows, 16), jnp.float32),
            pltpu.VMEM((CH, CHUNK), jnp.int32),
            pltpu.VMEM((CH, CHUNK), jnp.int32),
            pltpu.VMEM((CHUNK, 16), jnp.float32),
            pltpu.SemaphoreType.DMA,
        ],
        compiler_params=pltpu.CompilerParams(use_tc_tiling_on_sc=False),
    )
    def kern(table_hbm, gidx_hbm, sidx_hbm, zeros_hbm, out_hbm,
             acc, gidx_v, sidx_v, rows_v, sem):
        c = lax.axis_index("c")
        s = lax.axis_index("s")
        wid = c * NS + s
        pltpu.sync_copy(zeros_hbm.at[pl.ds(s * stripe, stripe)],
                        acc.at[pl.ds(s * stripe, stripe)])
        pltpu.sync_copy(gidx_hbm.at[wid], gidx_v)
        pltpu.sync_copy(sidx_hbm.at[wid], sidx_v)
        plsc.subcore_barrier()

        def body(j, carry):
            pltpu.async_copy(table_hbm.at[gidx_v.at[j]], rows_v, sem).wait()
            pltpu.sync_copy(rows_v, acc.at[sidx_v.at[j]], add=True)
            return carry

        lax.fori_loop(0, CH, body, 0)
        plsc.subcore_barrier()
        pltpu.sync_copy(acc.at[pl.ds(s * stripe, stripe)],
                        out_hbm.at[c, pl.ds(s * stripe, stripe)])

    return kern(table, gidx, sidx, zeros)


def _count_pass(sidx, ones, zeros):
    stripe = S_ROWS // NS

    @functools.partial(
        pl.kernel,
        out_type=jax.ShapeDtypeStruct((NC, S_ROWS, 16), jnp.float32),
        mesh=_sc_mesh(),
        scratch_types=[
            pltpu.VMEM_SHARED((S_ROWS, 16), jnp.float32),
            pltpu.VMEM((CH, CHUNK), jnp.int32),
            pltpu.VMEM((CHUNK, 16), jnp.float32),
        ],
        compiler_params=pltpu.CompilerParams(use_tc_tiling_on_sc=False),
    )
    def kern(sidx_hbm, ones_hbm, zeros_hbm, out_hbm, acc, sidx_v, rows_v):
        c = lax.axis_index("c")
        s = lax.axis_index("s")
        wid = c * NS + s
        pltpu.sync_copy(zeros_hbm.at[pl.ds(s * stripe, stripe)],
                        acc.at[pl.ds(s * stripe, stripe)])
        pltpu.sync_copy(sidx_hbm.at[wid], sidx_v)
        pltpu.sync_copy(ones_hbm, rows_v)
        plsc.subcore_barrier()

        def body(j, carry):
            pltpu.sync_copy(rows_v, acc.at[sidx_v.at[j]], add=True)
            return carry

        lax.fori_loop(0, CH, body, 0)
        plsc.subcore_barrier()
        pltpu.sync_copy(acc.at[pl.ds(s * stripe, stripe)],
                        out_hbm.at[c, pl.ds(s * stripe, stripe)])

    return kern(sidx, ones, zeros)


def _prep_kernel(x_ref, wcat_ref, root_ref, b_ref, xw_ref, base_ref):
    x = x_ref[...]
    xw_ref[...] = jnp.dot(x, wcat_ref[...], preferred_element_type=jnp.float32)
    base_ref[...] = (
        jnp.dot(x, root_ref[...], preferred_element_type=jnp.float32)
        + b_ref[...]
    )


def _prep(x, wcat, root, b, d_in, blk):
    grid = N // blk
    return pl.pallas_call(
        _prep_kernel,
        grid=(grid,),
        in_specs=[
            pl.BlockSpec((blk, d_in), lambda i: (i, 0)),
            pl.BlockSpec((d_in, R * H), lambda i: (0, 0)),
            pl.BlockSpec((d_in, H), lambda i: (0, 0)),
            pl.BlockSpec((1, H), lambda i: (0, 0)),
        ],
        out_specs=[
            pl.BlockSpec((blk, R * H), lambda i: (i, 0)),
            pl.BlockSpec((blk, H), lambda i: (i, 0)),
        ],
        out_shape=[
            jax.ShapeDtypeStruct((N, R * H), jnp.float32),
            jax.ShapeDtypeStruct((N, H), jnp.float32),
        ],
    )(x, wcat, root, b)


def _agg_message(sa_ref, sb_ref, ca_ref, cb_ref):
    svals = sa_ref[...] + sb_ref[...]
    cnt = ca_ref[...] + cb_ref[...]
    msg = jnp.zeros(sa_ref.shape[:1] + (16,), jnp.float32)
    for r in range(R):
        inv = 1.0 / jnp.maximum(cnt[:, r * 16 : r * 16 + 1], 1.0)
        msg = msg + svals[:, r * 16 : (r + 1) * 16] * inv
    return msg


def _mid_kernel(sa_ref, sb_ref, ca_ref, cb_ref, base_ref, wcat_ref,
                root_ref, b_ref, hw_ref, base2_ref):
    h = jax.nn.relu(base_ref[...] + _agg_message(sa_ref, sb_ref, ca_ref, cb_ref))
    hw_ref[...] = jnp.dot(h, wcat_ref[...], preferred_element_type=jnp.float32)
    base2_ref[...] = (
        jnp.dot(h, root_ref[...], preferred_element_type=jnp.float32)
        + b_ref[...]
    )


def _final_kernel(sa_ref, sb_ref, ca_ref, cb_ref, base_ref, out_ref):
    out_ref[...] = base_ref[...] + _agg_message(sa_ref, sb_ref, ca_ref, cb_ref)


def kernel(x, edge_index, edge_type, W1, root1, b1, W2, root2, b2):
    src = edge_index[0].astype(jnp.int32)
    dst = edge_index[1].astype(jnp.int32)
    et = edge_type.astype(jnp.int32)

    gidx = src * R + et
    sidx = dst * R + et
    pad = E_PAD - E
    gidx = jnp.concatenate([gidx, jnp.zeros((pad,), jnp.int32)])
    sidx = jnp.concatenate([sidx, jnp.full((pad,), S_TRASH, jnp.int32)])
    gidx = gidx.reshape(NW, CH, CHUNK)
    sidx = sidx.reshape(NW, CH, CHUNK)

    wcat1 = W1.transpose(1, 0, 2).reshape(D_IN, R * H).astype(jnp.float32)
    wcat2 = W2.transpose(1, 0, 2).reshape(H, R * C_OUT).astype(jnp.float32)

    zeros_s = jnp.zeros((S_ROWS, 16), jnp.float32)
    ones_c = jnp.ones((CHUNK, 16), jnp.float32)

    cacc = _count_pass(sidx, ones_c, zeros_s)
    ca = cacc[0, : N * R, :].reshape(N, R * 16)
    cb = cacc[1, : N * R, :].reshape(N, R * 16)

    xw1, base1 = _prep(x, wcat1, root1.astype(jnp.float32),
                       b1.reshape(1, H).astype(jnp.float32), D_IN, 1000)
    s1 = _edge_pass(xw1.reshape(N * R, 16), gidx, sidx, zeros_s, S_ROWS)
    s1a = s1[0, : N * R, :].reshape(N, R * 16)
    s1b = s1[1, : N * R, :].reshape(N, R * 16)

    blk = 1000
    full = lambda shape: pl.BlockSpec(shape, lambda i: (0, 0))
    rowblk = lambda w: pl.BlockSpec((blk, w), lambda i: (i, 0))

    hw2, base2 = pl.pallas_call(
        _mid_kernel,
        grid=(N // blk,),
        in_specs=[
            rowblk(R * 16), rowblk(R * 16), rowblk(R * 16), rowblk(R * 16),
            rowblk(H), full((H, R * C_OUT)), full((H, C_OUT)), full((1, C_OUT)),
        ],
        out_specs=[rowblk(R * C_OUT), rowblk(C_OUT)],
        out_shape=[
            jax.ShapeDtypeStruct((N, R * C_OUT), jnp.float32),
            jax.ShapeDtypeStruct((N, C_OUT), jnp.float32),
        ],
    )(s1a, s1b, ca, cb, base1, wcat2, root2.astype(jnp.float32),
      b2.reshape(1, C_OUT).astype(jnp.float32))

    s2 = _edge_pass(hw2.reshape(N * R, 16), gidx, sidx, zeros_s, S_ROWS)
    s2a = s2[0, : N * R, :].reshape(N, R * 16)
    s2b = s2[1, : N * R, :].reshape(N, R * 16)

    out = pl.pallas_call(
        _final_kernel,
        grid=(N // blk,),
        in_specs=[rowblk(R * 16), rowblk(R * 16), rowblk(R * 16),
                  rowblk(R * 16), rowblk(C_OUT)],
        out_specs=rowblk(C_OUT),
        out_shape=jax.ShapeDtypeStruct((N, C_OUT), jnp.float32),
    )(s2a, s2b, ca, cb, base2)

    return out
